# baseline (speedup 1.0000x reference)
import functools

import jax
import jax.numpy as jnp
from jax import lax
from jax.experimental import pallas as pl
from jax.experimental.pallas import tpu as pltpu
from jax.experimental.pallas import tpu_sc as plsc

L = 16


def kernel(u_idx, i_idx, u_emb, i_emb, u_bias, i_bias):
    B = u_idx.shape[0]
    F = u_emb.shape[1]
    info = plsc.get_sparse_core_info()
    NC, NS = info.num_cores, info.num_subcores
    NW = NC * NS
    b_per_w = B // NW

    mesh = plsc.VectorSubcoreMesh(core_axis_name="c", subcore_axis_name="s")

    @functools.partial(
        pl.kernel,
        mesh=mesh,
        out_type=jax.ShapeDtypeStruct((B,), jnp.float32),
        compiler_params=pltpu.CompilerParams(
            needs_layout_passes=False, use_tc_tiling_on_sc=False),
        scratch_types=[
            pltpu.VMEM((b_per_w,), jnp.int32),
            pltpu.VMEM((b_per_w,), jnp.int32),
            pltpu.VMEM((b_per_w, F), jnp.float32),
            pltpu.VMEM((b_per_w, F), jnp.float32),
            pltpu.VMEM((b_per_w,), jnp.float32),
            pltpu.VMEM((b_per_w,), jnp.float32),
            pltpu.VMEM((b_per_w,), jnp.float32),
            pltpu.SemaphoreType.DMA,
        ],
    )
    def sc_kernel(u_idx_hbm, i_idx_hbm, u_emb_hbm, i_emb_hbm, ub_hbm, ib_hbm,
                  out_hbm, uidx_v, iidx_v, urows_v, irows_v, ub_v, ib_v,
                  out_v, sem):
        wid = lax.axis_index("s") * NC + lax.axis_index("c")
        base = wid * b_per_w
        pltpu.sync_copy(u_idx_hbm.at[pl.ds(base, b_per_w)], uidx_v)
        pltpu.sync_copy(i_idx_hbm.at[pl.ds(base, b_per_w)], iidx_v)
        cu = pltpu.async_copy(u_emb_hbm.at[uidx_v], urows_v, sem)
        ci = pltpu.async_copy(i_emb_hbm.at[iidx_v], irows_v, sem)
        cub = pltpu.async_copy(ub_hbm.at[uidx_v], ub_v, sem)
        cib = pltpu.async_copy(ib_hbm.at[iidx_v], ib_v, sem)
        cu.wait()
        ci.wait()
        cub.wait()
        cib.wait()

        def body(g, carry):
            rows = lax.iota(jnp.int32, L) + g * L
            acc = ub_v[pl.ds(g * L, L)] + ib_v[pl.ds(g * L, L)]
            for f in range(F):
                cols = jnp.full((L,), f, jnp.int32)
                uv = plsc.load_gather(urows_v, [rows, cols])
                iv = plsc.load_gather(irows_v, [rows, cols])
                acc = acc + uv * iv
            out_v[pl.ds(g * L, L)] = acc
            return carry

        lax.fori_loop(0, b_per_w // L, body, 0)
        pltpu.sync_copy(out_v, out_hbm.at[pl.ds(base, b_per_w)])

    return sc_kernel(u_idx, i_idx, u_emb, i_emb,
                     u_bias.reshape(-1), i_bias.reshape(-1))
